# baseline (speedup 1.0000x reference)
import jax
import jax.numpy as jnp
from jax import lax
from jax.experimental import pallas as pl
from jax.experimental.pallas import tpu as pltpu
from jax.experimental.pallas import tpu_sc as plsc

G = 4096
FP = 64
D = 256
L = 16
NW = 32
GPW = G // NW
CD = D // L


def _grouper_sc(table_hbm, idx_hbm, m_hbm, out_hbm, idx_a, m_a, rows0,
                rows1, out_a, sem0, sem1):
    wid = lax.axis_index("s") * 2 + lax.axis_index("c")
    g0 = wid * GPW

    pltpu.sync_copy(idx_hbm.at[pl.ds(g0, GPW), :], idx_a)
    pltpu.sync_copy(m_hbm.at[pl.ds(g0 * L, GPW * L)], m_a)

    rows = (rows0, rows1)
    sems = (sem0, sem1)

    def count(t):
        return m_a[pl.ds(t * L, L)][0]

    def nblocks(t):
        return (count(t) + (L - 1)) // L

    def start(t, b):
        nb = nblocks(t)
        for k in range(FP // L):
            @pl.when(nb > k)
            def _(k=k):
                pltpu.async_copy(
                    table_hbm.at[idx_a.at[t, pl.ds(k * L, L)]],
                    rows[b].at[pl.ds(k * L, L), :], sems[b])

    def wait(t, b):
        nb = nblocks(t)
        for k in range(FP // L):
            @pl.when(nb > k)
            def _(k=k):
                pltpu.make_async_copy(
                    table_hbm.at[idx_a.at[0, pl.ds(0, L)]],
                    rows[b].at[pl.ds(k * L, L), :], sems[b]).wait()

    def reduce_group(t, b):
        rows_b = rows[b]
        mt = count(t)
        nfull = mt // L

        def block_body(k, a):
            a = list(a)
            base = k * L
            for fi in range(L):
                for c in range(CD):
                    a[c] = a[c] + rows_b[base + fi, pl.ds(c * L, L)]
            return tuple(a)

        def tail_body(f, a):
            a = list(a)
            for c in range(CD):
                a[c] = a[c] + rows_b[f, pl.ds(c * L, L)]
            return tuple(a)

        zeros = tuple(jnp.zeros((L,), jnp.float32) for _ in range(CD))
        acc = lax.fori_loop(0, nfull, block_body, zeros)
        acc = lax.fori_loop(nfull * L, mt, tail_body, acc)
        for c in range(CD):
            out_a[pl.ds(t * D + c * L, L)] = acc[c]

    start(0, 0)

    def body(tt, carry):
        t0 = tt * 2
        start(t0 + 1, 1)
        wait(t0, 0)
        reduce_group(t0, 0)

        @pl.when(t0 + 2 < GPW)
        def _():
            start(t0 + 2, 0)

        wait(t0 + 1, 1)
        reduce_group(t0 + 1, 1)
        return carry

    lax.fori_loop(0, GPW // 2, body, 0)
    pltpu.sync_copy(out_a, out_hbm.at[pl.ds(g0 * D, GPW * D)])


def kernel(in_features, W, grp_edge_feat, edge_to_node, grp_edge_idx_plus,
           grp_num_feat, grp_feat_idx_plus):
    ratio = 1.0 / grp_num_feat.astype(jnp.float32)
    csum = jnp.cumsum(
        jnp.broadcast_to(ratio[:, None], (G, FP)), axis=1)
    hard = csum <= 1.0
    m = jnp.sum(hard, axis=1).astype(jnp.int32)
    m_rep = jnp.repeat(m, L)

    idx2d = grp_feat_idx_plus.astype(jnp.int32)

    mesh = plsc.VectorSubcoreMesh(core_axis_name="c", subcore_axis_name="s")
    run = pl.kernel(
        _grouper_sc,
        out_type=jax.ShapeDtypeStruct((G * D,), jnp.float32),
        mesh=mesh,
        scratch_types=[
            pltpu.VMEM((GPW, FP), jnp.int32),
            pltpu.VMEM((GPW * L,), jnp.int32),
            pltpu.VMEM((FP, D), jnp.float32),
            pltpu.VMEM((FP, D), jnp.float32),
            pltpu.VMEM((GPW * D,), jnp.float32),
            pltpu.SemaphoreType.DMA,
            pltpu.SemaphoreType.DMA,
        ],
    )
    return run(in_features, idx2d, m_rep).reshape(G, D)


# speedup vs baseline: 1.0431x; 1.0431x over previous
import jax
import jax.numpy as jnp
from jax import lax
from jax.experimental import pallas as pl
from jax.experimental.pallas import tpu as pltpu
from jax.experimental.pallas import tpu_sc as plsc

G = 4096
FP = 64
D = 256
L = 16
NW = 32
GPW = G // NW
CD = D // L


def _grouper_sc(table_hbm, idx_hbm, m_hbm, out_hbm, idx_a, m_a, rows0,
                rows1, out_a, sem0, sem1):
    wid = lax.axis_index("s") * 2 + lax.axis_index("c")
    g0 = wid * GPW

    pltpu.sync_copy(idx_hbm.at[pl.ds(g0, GPW), :], idx_a)
    pltpu.sync_copy(m_hbm.at[pl.ds(g0 * L, GPW * L)], m_a)

    rows = (rows0, rows1)
    sems = (sem0, sem1)

    def count(t):
        return m_a[pl.ds(t * L, L)][0]

    def nblocks(t):
        return (count(t) + (L - 1)) // L

    def start(t, b):
        pltpu.async_copy(table_hbm.at[idx_a.at[t]], rows[b], sems[b])

    def wait(t, b):
        pltpu.make_async_copy(table_hbm.at[idx_a.at[0]], rows[b],
                              sems[b]).wait()

    def reduce_group(t, b):
        rows_b = rows[b]
        mt = count(t)
        nfull = mt // L

        def block_body(k, a):
            a = list(a)
            base = k * L
            for fi in range(L):
                for c in range(CD):
                    a[c] = a[c] + rows_b[base + fi, pl.ds(c * L, L)]
            return tuple(a)

        def tail_body(f, a):
            a = list(a)
            for c in range(CD):
                a[c] = a[c] + rows_b[f, pl.ds(c * L, L)]
            return tuple(a)

        zeros = tuple(jnp.zeros((L,), jnp.float32) for _ in range(CD))
        acc = lax.fori_loop(0, nfull, block_body, zeros)
        acc = lax.fori_loop(nfull * L, mt, tail_body, acc)
        for c in range(CD):
            out_a[pl.ds(t * D + c * L, L)] = acc[c]

    start(0, 0)

    def body(tt, carry):
        t0 = tt * 2
        start(t0 + 1, 1)
        wait(t0, 0)
        reduce_group(t0, 0)

        @pl.when(t0 + 2 < GPW)
        def _():
            start(t0 + 2, 0)

        wait(t0 + 1, 1)
        reduce_group(t0 + 1, 1)
        return carry

    lax.fori_loop(0, GPW // 2, body, 0)
    pltpu.sync_copy(out_a, out_hbm.at[pl.ds(g0 * D, GPW * D)])


def kernel(in_features, W, grp_edge_feat, edge_to_node, grp_edge_idx_plus,
           grp_num_feat, grp_feat_idx_plus):
    ratio = 1.0 / grp_num_feat.astype(jnp.float32)
    csum = jnp.cumsum(
        jnp.broadcast_to(ratio[:, None], (G, FP)), axis=1)
    hard = csum <= 1.0
    m = jnp.sum(hard, axis=1).astype(jnp.int32)
    m_rep = jnp.repeat(m, L)

    idx2d = grp_feat_idx_plus.astype(jnp.int32)

    mesh = plsc.VectorSubcoreMesh(core_axis_name="c", subcore_axis_name="s")
    run = pl.kernel(
        _grouper_sc,
        out_type=jax.ShapeDtypeStruct((G * D,), jnp.float32),
        mesh=mesh,
        scratch_types=[
            pltpu.VMEM((GPW, FP), jnp.int32),
            pltpu.VMEM((GPW * L,), jnp.int32),
            pltpu.VMEM((FP, D), jnp.float32),
            pltpu.VMEM((FP, D), jnp.float32),
            pltpu.VMEM((GPW * D,), jnp.float32),
            pltpu.SemaphoreType.DMA,
            pltpu.SemaphoreType.DMA,
        ],
    )
    return run(in_features, idx2d, m_rep).reshape(G, D)


# speedup vs baseline: 1.5818x; 1.5165x over previous
import jax
import jax.numpy as jnp
from jax import lax
from jax.experimental import pallas as pl
from jax.experimental.pallas import tpu as pltpu
from jax.experimental.pallas import tpu_sc as plsc

G = 4096
FP = 64
D = 256
L = 16
NW = 32
GPW = G // NW
CD = D // L
GC = 2
NCH = GPW // GC


def _grouper_sc(table_hbm, idx_hbm, m_hbm, out_hbm, idx_a, m_a, rows0,
                rows1, out_a, sem0, sem1):
    wid = lax.axis_index("s") * 2 + lax.axis_index("c")
    g0 = wid * GPW

    pltpu.sync_copy(idx_hbm.at[pl.ds(g0 * FP, GPW * FP)], idx_a)
    pltpu.sync_copy(m_hbm.at[pl.ds(g0 * L, GPW * L)], m_a)

    rows = (rows0, rows1)
    sems = (sem0, sem1)

    def start(ch, b):
        pltpu.async_copy(
            table_hbm.at[idx_a.at[pl.ds(ch * GC * FP, GC * FP)]],
            rows[b], sems[b])

    def wait(b):
        pltpu.make_async_copy(
            table_hbm.at[idx_a.at[pl.ds(0, GC * FP)]], rows[b],
            sems[b]).wait()

    def reduce_chunk(ch, b):
        rows_b = rows[b]
        for j in range(GC):
            t = ch * GC + j
            mt = m_a[pl.ds(t * L, L)][0]

            def fbody(f, a, j=j):
                a = list(a)
                for c in range(CD):
                    a[c] = a[c] + rows_b[j * FP + f, pl.ds(c * L, L)]
                return tuple(a)

            acc = lax.fori_loop(
                0, mt, fbody,
                tuple(jnp.zeros((L,), jnp.float32) for _ in range(CD)))
            for c in range(CD):
                out_a[pl.ds(t * D + c * L, L)] = acc[c]

    start(0, 0)

    def body(cc, carry):
        c0 = cc * 2
        start(c0 + 1, 1)
        wait(0)
        reduce_chunk(c0, 0)

        @pl.when(c0 + 2 < NCH)
        def _():
            start(c0 + 2, 0)

        wait(1)
        reduce_chunk(c0 + 1, 1)
        return carry

    lax.fori_loop(0, NCH // 2, body, 0)
    pltpu.sync_copy(out_a, out_hbm.at[pl.ds(g0 * D, GPW * D)])


def kernel(in_features, W, grp_edge_feat, edge_to_node, grp_edge_idx_plus,
           grp_num_feat, grp_feat_idx_plus):
    ratio = 1.0 / grp_num_feat.astype(jnp.float32)
    csum = jnp.cumsum(
        jnp.broadcast_to(ratio[:, None], (G, FP)), axis=1)
    hard = csum <= 1.0
    m = jnp.sum(hard, axis=1).astype(jnp.int32)
    m_rep = jnp.repeat(m, L)

    idx_flat = grp_feat_idx_plus.reshape(-1).astype(jnp.int32)

    mesh = plsc.VectorSubcoreMesh(core_axis_name="c", subcore_axis_name="s")
    run = pl.kernel(
        _grouper_sc,
        out_type=jax.ShapeDtypeStruct((G * D,), jnp.float32),
        mesh=mesh,
        scratch_types=[
            pltpu.VMEM((GPW * FP,), jnp.int32),
            pltpu.VMEM((GPW * L,), jnp.int32),
            pltpu.VMEM((GC * FP, D), jnp.float32),
            pltpu.VMEM((GC * FP, D), jnp.float32),
            pltpu.VMEM((GPW * D,), jnp.float32),
            pltpu.SemaphoreType.DMA,
            pltpu.SemaphoreType.DMA,
        ],
    )
    return run(in_features, idx_flat, m_rep).reshape(G, D)


# speedup vs baseline: 1.8227x; 1.1523x over previous
import jax
import jax.numpy as jnp
from jax import lax
from jax.experimental import pallas as pl
from jax.experimental.pallas import tpu as pltpu
from jax.experimental.pallas import tpu_sc as plsc

G = 4096
FP = 64
D = 256
L = 16
NW = 32
GPW = G // NW
CD = D // L
GC = 2
NCH = GPW // GC


def _grouper_sc(table_hbm, idx_hbm, m_hbm, out_hbm, idx_a, m_a, rows0,
                rows1, out_a, sem0, sem1):
    wid = lax.axis_index("s") * 2 + lax.axis_index("c")
    g0 = wid * GPW

    pltpu.sync_copy(idx_hbm.at[pl.ds(g0 * FP, GPW * FP)], idx_a)
    pltpu.sync_copy(m_hbm.at[pl.ds(g0 * L, GPW * L)], m_a)

    rows = (rows0, rows1)
    sems = (sem0, sem1)

    H = FP // 2

    def start(ch, b):
        for j in range(GC):
            t = ch * GC + j
            mt = m_a[pl.ds(t * L, L)][0]
            pltpu.async_copy(
                table_hbm.at[idx_a.at[pl.ds(t * FP, H)]],
                rows[b].at[pl.ds(j * FP, H), :], sems[b])

            @pl.when(mt > H)
            def _(j=j, t=t):
                pltpu.async_copy(
                    table_hbm.at[idx_a.at[pl.ds(t * FP + H, H)]],
                    rows[b].at[pl.ds(j * FP + H, H), :], sems[b])

    def wait(ch, b):
        for j in range(GC):
            t = ch * GC + j
            mt = m_a[pl.ds(t * L, L)][0]
            pltpu.make_async_copy(
                table_hbm.at[idx_a.at[pl.ds(0, H)]],
                rows[b].at[pl.ds(j * FP, H), :], sems[b]).wait()

            @pl.when(mt > H)
            def _(j=j):
                pltpu.make_async_copy(
                    table_hbm.at[idx_a.at[pl.ds(0, H)]],
                    rows[b].at[pl.ds(j * FP + H, H), :], sems[b]).wait()

    def reduce_chunk(ch, b):
        rows_b = rows[b]
        for j in range(GC):
            t = ch * GC + j
            mt = m_a[pl.ds(t * L, L)][0]

            def fbody(f, a, j=j):
                a = list(a)
                for c in range(CD):
                    a[c] = a[c] + rows_b[j * FP + f, pl.ds(c * L, L)]
                return tuple(a)

            acc = lax.fori_loop(
                0, mt, fbody,
                tuple(jnp.zeros((L,), jnp.float32) for _ in range(CD)))
            for c in range(CD):
                out_a[pl.ds(t * D + c * L, L)] = acc[c]

    start(0, 0)

    def body(cc, carry):
        c0 = cc * 2
        start(c0 + 1, 1)
        wait(c0, 0)
        reduce_chunk(c0, 0)

        @pl.when(c0 + 2 < NCH)
        def _():
            start(c0 + 2, 0)

        wait(c0 + 1, 1)
        reduce_chunk(c0 + 1, 1)
        return carry

    lax.fori_loop(0, NCH // 2, body, 0)
    pltpu.sync_copy(out_a, out_hbm.at[pl.ds(g0 * D, GPW * D)])


def kernel(in_features, W, grp_edge_feat, edge_to_node, grp_edge_idx_plus,
           grp_num_feat, grp_feat_idx_plus):
    ratio = 1.0 / grp_num_feat.astype(jnp.float32)
    csum = jnp.cumsum(
        jnp.broadcast_to(ratio[:, None], (G, FP)), axis=1)
    hard = csum <= 1.0
    m = jnp.sum(hard, axis=1).astype(jnp.int32)
    m_rep = jnp.repeat(m, L)

    idx_flat = grp_feat_idx_plus.reshape(-1).astype(jnp.int32)

    mesh = plsc.VectorSubcoreMesh(core_axis_name="c", subcore_axis_name="s")
    run = pl.kernel(
        _grouper_sc,
        out_type=jax.ShapeDtypeStruct((G * D,), jnp.float32),
        mesh=mesh,
        scratch_types=[
            pltpu.VMEM((GPW * FP,), jnp.int32),
            pltpu.VMEM((GPW * L,), jnp.int32),
            pltpu.VMEM((GC * FP, D), jnp.float32),
            pltpu.VMEM((GC * FP, D), jnp.float32),
            pltpu.VMEM((GPW * D,), jnp.float32),
            pltpu.SemaphoreType.DMA,
            pltpu.SemaphoreType.DMA,
        ],
    )
    return run(in_features, idx_flat, m_rep).reshape(G, D)
